# baseline (speedup 1.0000x reference)
import jax
import jax.numpy as jnp
from jax.experimental import pallas as pl

_VOCAB = 100000
_DIM = 64
_ROWS = (_VOCAB * _DIM) // 128
_BLOCK = 5000


def _copy_body(x_ref, o_ref):
    o_ref[...] = x_ref[...]


def kernel(embeddings):
    x = embeddings.reshape(_ROWS, 128)
    out = pl.pallas_call(
        _copy_body,
        grid=(_ROWS // _BLOCK,),
        in_specs=[pl.BlockSpec((_BLOCK, 128), lambda i: (i, 0))],
        out_specs=pl.BlockSpec((_BLOCK, 128), lambda i: (i, 0)),
        out_shape=jax.ShapeDtypeStruct((_ROWS, 128), jnp.float32),
    )(x)
    return out.reshape(_VOCAB, _DIM)
